# baseline (speedup 1.0000x reference)
import functools
import math

import jax
import jax.numpy as jnp
from jax import lax
from jax.experimental import pallas as pl
from jax.experimental.pallas import tpu as pltpu
from jax.experimental.pallas import tpu_sc as plsc

VOCAB = 1000000
EMBED = 64
MAXLEN = 200
PAD = 0

NC = 2
NS = 16
NW = NC * NS

LANES = 16
IDX_PER_STREAM = 40
STREAMS_PER_CHUNK = 10
ROWS = 20480
ROWS_PER_WORKER = ROWS // NW
CHUNKS = ROWS_PER_WORKER // STREAMS_PER_CHUNK


def _make_pos_embed(max_length, embed_size):
    t = jnp.arange(1, max_length + 1, dtype=jnp.float32)
    omega = jnp.arange(1, embed_size // 2 + 1, dtype=jnp.float32) / embed_size
    wt = t[:, None] * jnp.power(10000.0, -omega)[None, :]
    pos = jnp.zeros((max_length, embed_size), dtype=jnp.float32)
    pos = pos.at[:, 0::2].set(jnp.sin(wt))
    pos = pos.at[:, 1::2].set(jnp.cos(wt))
    return pos


def _body(x_hbm, table_hbm, pos_hbm, out_hbm, idx_flat, buf, pos_v, sem):
    wid = lax.axis_index("s") * NC + lax.axis_index("c")
    chunk_idx = STREAMS_PER_CHUNK * IDX_PER_STREAM

    pltpu.sync_copy(pos_hbm, pos_v)

    def chunk_body(c, carry):
        rowb = wid * ROWS_PER_WORKER + c * STREAMS_PER_CHUNK

        pltpu.sync_copy(x_hbm.at[pl.ds(rowb * IDX_PER_STREAM, chunk_idx)],
                        idx_flat)

        handles = [
            pltpu.async_copy(
                table_hbm.at[idx_flat.at[pl.ds(q * IDX_PER_STREAM,
                                               IDX_PER_STREAM)]],
                buf.at[q], sem)
            for q in range(STREAMS_PER_CHUNK)
        ]
        for h in handles:
            h.wait()

        for q in range(STREAMS_PER_CHUNK):
            poff = (q * IDX_PER_STREAM) % MAXLEN

            def row_body(r, carry2, q=q, poff=poff):
                fr = q * IDX_PER_STREAM + r
                ivec = plsc.load_gather(idx_flat, [lax.broadcast(fr, (LANES,))])
                s = jnp.where(ivec == PAD, jnp.float32(0.0),
                              jnp.float32(math.sqrt(float(EMBED))))
                for e in range(EMBED // LANES):
                    sl = pl.ds(e * LANES, LANES)
                    buf[q, r, sl] = buf[q, r, sl] * s + pos_v[poff + r, sl]
                return carry2

            lax.fori_loop(0, IDX_PER_STREAM, row_body, 0, unroll=2)

        pltpu.sync_copy(buf, out_hbm.at[pl.ds(rowb, STREAMS_PER_CHUNK)])
        return carry

    lax.fori_loop(0, CHUNKS, chunk_body, 0)


@functools.lru_cache(maxsize=None)
def _emb_call():
    return functools.partial(
        pl.kernel,
        out_type=jax.ShapeDtypeStruct((ROWS, IDX_PER_STREAM, EMBED),
                                      jnp.float32),
        mesh=plsc.VectorSubcoreMesh(
            core_axis_name="c", subcore_axis_name="s",
            num_cores=NC, num_subcores=NS),
        scratch_types=[
            pltpu.VMEM((STREAMS_PER_CHUNK * IDX_PER_STREAM,), jnp.int32),
            pltpu.VMEM((STREAMS_PER_CHUNK, IDX_PER_STREAM, EMBED),
                       jnp.float32),
            pltpu.VMEM((MAXLEN, EMBED), jnp.float32),
            pltpu.SemaphoreType.DMA,
        ],
        compiler_params=pltpu.CompilerParams(use_tc_tiling_on_sc=False,
                                             needs_layout_passes=False),
    )(_body)


def kernel(x, table):
    B, L = x.shape
    x2 = x.reshape(ROWS * IDX_PER_STREAM).astype(jnp.int32)
    pos = _make_pos_embed(MAXLEN, EMBED)[:L]
    out = _emb_call()(x2, table, pos)
    return out.reshape(B, L, EMBED)
